# baseline (speedup 1.0000x reference)
import jax
import jax.numpy as jnp
from jax import lax
from jax.experimental import pallas as pl
from jax.experimental.pallas import tpu as pltpu
from jax.experimental.pallas import tpu_sc as plsc

N_ENVS = 4096
H = 64
W = 64
VIEW = 11
NC = 2
NS = 16
LANES = 16
NW = NC * NS
EPW = N_ENVS // NW
G = EPW // LANES
OUT_PER_ENV = 5 * VIEW * VIEW
PATCH = VIEW * VIEW


def _sc_body(grids_hbm, en_hbm, act_hbm, ax_hbm, ay_hbm, dx_hbm, dy_hbm,
             out_hbm, slab_v, out_v, eidx_v, en_v, act_v, ax_v, ay_v, dx_v,
             dy_v, sem):
    wid = lax.axis_index("s") * NC + lax.axis_index("c")
    base = pl.multiple_of(wid * EPW, EPW)

    pltpu.sync_copy(en_hbm.at[pl.ds(base, EPW)], en_v)
    pltpu.sync_copy(act_hbm.at[pl.ds(base, EPW)], act_v)
    pltpu.sync_copy(ax_hbm.at[pl.ds(base, EPW)], ax_v)
    pltpu.sync_copy(ay_hbm.at[pl.ds(base, EPW)], ay_v)
    pltpu.sync_copy(dx_hbm, dx_v)
    pltpu.sync_copy(dy_hbm, dy_v)

    lane = lax.iota(jnp.int32, LANES)

    @pl.loop(0, G)
    def _(g):
        off = g * LANES
        e0 = pl.multiple_of(base + off, LANES)
        eidx_v[...] = e0 + lane
        cp = pltpu.async_copy(grids_hbm.at[eidx_v], slab_v, sem)

        agx = ax_v[pl.ds(off, LANES)]
        agy = ay_v[pl.ds(off, LANES)]
        acts = act_v[pl.ds(off, LANES)]
        en = en_v[pl.ds(off, LANES)]

        ax = jnp.clip(agx, 1, W - 2)
        ay = jnp.clip(agy, 1, H - 2)
        dx = plsc.load_gather(dx_v, [acts])
        dy = plsc.load_gather(dy_v, [acts])
        nx = jnp.clip(ax + dx, 1, W - 2)
        ny = jnp.clip(ay + dy, 1, H - 2)
        cp.wait()

        tcf = plsc.load_gather(slab_v, [lane, ny, nx])
        blocked = (tcf * 4.0).astype(jnp.int32) == 1
        fx = jnp.where(blocked, ax, nx)
        fy = jnp.where(blocked, ay, ny)
        cur = plsc.load_gather(slab_v, [lane, fy, fx])
        lc = (cur * 4.0).astype(jnp.int32)
        food = lc == 2
        poison = lc == 3
        reward = jnp.where(food, 10.0, 0.0) - jnp.where(poison, 20.0, 0.0) - 0.1
        enc = (en + reward) / 100.0
        centerval = jnp.where(food | poison, 0.0, cur)

        col0 = fx - 5

        @pl.loop(0, VIEW)
        def _(k):
            row = fy + (k - 5)
            rin = (row >= 0) & (row <= H - 1)
            rowc = jnp.clip(row, 0, H - 1)
            for j in range(VIEW):
                p = k * VIEW + j
                col = col0 + j
                inb = rin & (col >= 0) & (col <= W - 1)
                colc = jnp.clip(col, 0, W - 1)
                v = plsc.load_gather(slab_v, [lane, rowc, colc])
                patch = jnp.where(inb, v, 0.3)
                cell = (patch * 4.0).astype(jnp.int32)
                out_v[p, :] = patch
                out_v[PATCH + p, :] = jnp.where(cell == 1, 1.0, 0.0)
                out_v[2 * PATCH + p, :] = jnp.where(cell == 2, 1.0, 0.0)
                out_v[3 * PATCH + p, :] = jnp.where(cell == 3, 1.0, 0.0)
                out_v[4 * PATCH + p, :] = enc

        ccell = (centerval * 4.0).astype(jnp.int32)
        oc = 5 * VIEW + 5
        out_v[oc, :] = centerval
        out_v[PATCH + oc, :] = jnp.where(ccell == 1, 1.0, 0.0)
        out_v[2 * PATCH + oc, :] = jnp.where(ccell == 2, 1.0, 0.0)
        out_v[3 * PATCH + oc, :] = jnp.where(ccell == 3, 1.0, 0.0)

        pltpu.sync_copy(out_v, out_hbm.at[:, pl.ds(e0, LANES)])


def kernel(grids, agent_energy, actions, agent_x, agent_y):
    dx16 = jnp.array([0, 0, 0, -1, 1, -1, -1, 1, 1, 0, 0, 0, 0, 0, 0, 0],
                     jnp.int32)
    dy16 = jnp.array([0, -1, 1, 0, 0, -1, 1, -1, 1, 0, 0, 0, 0, 0, 0, 0],
                     jnp.int32)

    sc_fn = pl.kernel(
        _sc_body,
        out_type=jax.ShapeDtypeStruct((OUT_PER_ENV, N_ENVS), jnp.float32),
        mesh=plsc.VectorSubcoreMesh(core_axis_name="c", subcore_axis_name="s"),
        compiler_params=pltpu.CompilerParams(needs_layout_passes=False,
                                             use_tc_tiling_on_sc=False),
        scratch_types=[
            pltpu.VMEM((LANES, H, W), jnp.float32),
            pltpu.VMEM((OUT_PER_ENV, LANES), jnp.float32),
            pltpu.VMEM((LANES,), jnp.int32),
            pltpu.VMEM((EPW,), jnp.float32),
            pltpu.VMEM((EPW,), jnp.int32),
            pltpu.VMEM((EPW,), jnp.int32),
            pltpu.VMEM((EPW,), jnp.int32),
            pltpu.VMEM((LANES,), jnp.int32),
            pltpu.VMEM((LANES,), jnp.int32),
            pltpu.SemaphoreType.DMA,
        ],
    )
    flat = sc_fn(grids, agent_energy,
                 actions.astype(jnp.int32),
                 agent_x.astype(jnp.int32),
                 agent_y.astype(jnp.int32),
                 dx16, dy16)
    return flat.reshape(5, VIEW, VIEW, N_ENVS).transpose(3, 0, 1, 2)


# speedup vs baseline: 1.0728x; 1.0728x over previous
import jax
import jax.numpy as jnp
from jax import lax
from jax.experimental import pallas as pl
from jax.experimental.pallas import tpu as pltpu
from jax.experimental.pallas import tpu_sc as plsc

N_ENVS = 4096
H = 64
W = 64
VIEW = 11
NC = 2
NS = 16
LANES = 16
NW = NC * NS
EPW = N_ENVS // NW
G = EPW // LANES
OUT_PER_ENV = 5 * VIEW * VIEW
PATCH = VIEW * VIEW


NROWS = 13
Y0MAX = H - NROWS


def _sc_body(grids_hbm, en_hbm, act_hbm, ax_hbm, ay_hbm, dx_hbm, dy_hbm,
             out_hbm, rows_v, out_v, idx_v, en_v, act_v, ax_v, ay_v, dx_v,
             dy_v, sem):
    wid = lax.axis_index("s") * NC + lax.axis_index("c")
    base = pl.multiple_of(wid * EPW, EPW)

    pltpu.sync_copy(en_hbm.at[pl.ds(base, EPW)], en_v)
    pltpu.sync_copy(act_hbm.at[pl.ds(base, EPW)], act_v)
    pltpu.sync_copy(ax_hbm.at[pl.ds(base, EPW)], ax_v)
    pltpu.sync_copy(ay_hbm.at[pl.ds(base, EPW)], ay_v)
    pltpu.sync_copy(dx_hbm, dx_v)
    pltpu.sync_copy(dy_hbm, dy_v)

    lane = lax.iota(jnp.int32, LANES)

    @pl.loop(0, G)
    def _(g):
        off = g * LANES
        e0 = pl.multiple_of(base + off, LANES)

        agx = ax_v[pl.ds(off, LANES)]
        agy = ay_v[pl.ds(off, LANES)]
        acts = act_v[pl.ds(off, LANES)]
        en = en_v[pl.ds(off, LANES)]

        ax = jnp.clip(agx, 1, W - 2)
        ay = jnp.clip(agy, 1, H - 2)
        dx = plsc.load_gather(dx_v, [acts])
        dy = plsc.load_gather(dy_v, [acts])
        nx = jnp.clip(ax + dx, 1, W - 2)
        ny = jnp.clip(ay + dy, 1, H - 2)

        for k in range(NROWS):
            plsc.store_scatter(idx_v, [lane * LANES + k],
                               jnp.clip(ay - 6 + k, 0, H - 1))
        copies = []
        for l in range(LANES):
            copies.append(
                pltpu.async_copy(
                    grids_hbm.at[e0 + l].at[idx_v.at[pl.ds(l * LANES,
                                                           NROWS)]],
                    rows_v.at[pl.ds(l * NROWS, NROWS)], sem))

        rowb0 = lane * NROWS + 6 - ay
        for cp in copies:
            cp.wait()

        tcf = plsc.load_gather(rows_v, [rowb0 + ny, nx])
        blocked = (tcf * 4.0).astype(jnp.int32) == 1
        fx = jnp.where(blocked, ax, nx)
        fy = jnp.where(blocked, ay, ny)
        cur = plsc.load_gather(rows_v, [rowb0 + fy, fx])
        lc = (cur * 4.0).astype(jnp.int32)
        food = lc == 2
        poison = lc == 3
        reward = jnp.where(food, 10.0, 0.0) - jnp.where(poison, 20.0, 0.0) - 0.1
        enc = (en + reward) / 100.0
        centerval = jnp.where(food | poison, 0.0, cur)

        col0 = fx - 5
        rowp0 = rowb0 + fy - 5

        @pl.loop(0, VIEW)
        def _(k):
            row = fy + (k - 5)
            rin = (row >= 0) & (row <= H - 1)
            rowb = jnp.clip(rowp0 + k, lane * NROWS,
                            lane * NROWS + NROWS - 1)
            for j in range(VIEW):
                p = k * VIEW + j
                col = col0 + j
                inb = rin & (col >= 0) & (col <= W - 1)
                colc = jnp.clip(col, 0, W - 1)
                v = plsc.load_gather(rows_v, [rowb, colc])
                patch = jnp.where(inb, v, 0.3)
                cell = (patch * 4.0).astype(jnp.int32)
                out_v[p, :] = patch
                out_v[PATCH + p, :] = jnp.where(cell == 1, 1.0, 0.0)
                out_v[2 * PATCH + p, :] = jnp.where(cell == 2, 1.0, 0.0)
                out_v[3 * PATCH + p, :] = jnp.where(cell == 3, 1.0, 0.0)
                out_v[4 * PATCH + p, :] = enc

        ccell = (centerval * 4.0).astype(jnp.int32)
        oc = 5 * VIEW + 5
        out_v[oc, :] = centerval
        out_v[PATCH + oc, :] = jnp.where(ccell == 1, 1.0, 0.0)
        out_v[2 * PATCH + oc, :] = jnp.where(ccell == 2, 1.0, 0.0)
        out_v[3 * PATCH + oc, :] = jnp.where(ccell == 3, 1.0, 0.0)

        pltpu.sync_copy(out_v, out_hbm.at[:, pl.ds(e0, LANES)])


def kernel(grids, agent_energy, actions, agent_x, agent_y):
    dx16 = jnp.array([0, 0, 0, -1, 1, -1, -1, 1, 1, 0, 0, 0, 0, 0, 0, 0],
                     jnp.int32)
    dy16 = jnp.array([0, -1, 1, 0, 0, -1, 1, -1, 1, 0, 0, 0, 0, 0, 0, 0],
                     jnp.int32)

    sc_fn = pl.kernel(
        _sc_body,
        out_type=jax.ShapeDtypeStruct((OUT_PER_ENV, N_ENVS), jnp.float32),
        mesh=plsc.VectorSubcoreMesh(core_axis_name="c", subcore_axis_name="s"),
        compiler_params=pltpu.CompilerParams(needs_layout_passes=False,
                                             use_tc_tiling_on_sc=False),
        scratch_types=[
            pltpu.VMEM((NROWS * LANES, W), jnp.float32),
            pltpu.VMEM((OUT_PER_ENV, LANES), jnp.float32),
            pltpu.VMEM((LANES * LANES,), jnp.int32),
            pltpu.VMEM((EPW,), jnp.float32),
            pltpu.VMEM((EPW,), jnp.int32),
            pltpu.VMEM((EPW,), jnp.int32),
            pltpu.VMEM((EPW,), jnp.int32),
            pltpu.VMEM((LANES,), jnp.int32),
            pltpu.VMEM((LANES,), jnp.int32),
            pltpu.SemaphoreType.DMA,
        ],
    )
    flat = sc_fn(grids, agent_energy,
                 actions.astype(jnp.int32),
                 agent_x.astype(jnp.int32),
                 agent_y.astype(jnp.int32),
                 dx16, dy16)
    return flat.reshape(5, VIEW, VIEW, N_ENVS).transpose(3, 0, 1, 2)


# speedup vs baseline: 1.5885x; 1.4807x over previous
import jax
import jax.numpy as jnp
from jax import lax
from jax.experimental import pallas as pl
from jax.experimental.pallas import tpu as pltpu
from jax.experimental.pallas import tpu_sc as plsc

N_ENVS = 4096
H = 64
W = 64
VIEW = 11
NC = 2
NS = 16
LANES = 16
NW = NC * NS
EPW = N_ENVS // NW
G = EPW // LANES
OUT_PER_ENV = 5 * VIEW * VIEW
PATCH = VIEW * VIEW


NROWS = 13
Y0MAX = H - NROWS


def _sc_body(grids_hbm, en_hbm, act_hbm, ax_hbm, ay_hbm, dx_hbm, dy_hbm,
             out_hbm, slab_v, out_v, en_v, act_v, ax_v, ay_v, dx_v,
             dy_v, sem):
    wid = lax.axis_index("s") * NC + lax.axis_index("c")
    base = pl.multiple_of(wid * EPW, EPW)

    pltpu.sync_copy(en_hbm.at[pl.ds(base, EPW)], en_v)
    pltpu.sync_copy(act_hbm.at[pl.ds(base, EPW)], act_v)
    pltpu.sync_copy(ax_hbm.at[pl.ds(base, EPW)], ax_v)
    pltpu.sync_copy(ay_hbm.at[pl.ds(base, EPW)], ay_v)
    pltpu.sync_copy(dx_hbm, dx_v)
    pltpu.sync_copy(dy_hbm, dy_v)

    lane = lax.iota(jnp.int32, LANES)

    @pl.loop(0, G)
    def _(g):
        off = g * LANES
        e0 = pl.multiple_of(base + off, LANES)

        agx = ax_v[pl.ds(off, LANES)]
        agy = ay_v[pl.ds(off, LANES)]
        acts = act_v[pl.ds(off, LANES)]
        en = en_v[pl.ds(off, LANES)]

        ax = jnp.clip(agx, 1, W - 2)
        ay = jnp.clip(agy, 1, H - 2)
        dx = plsc.load_gather(dx_v, [acts])
        dy = plsc.load_gather(dy_v, [acts])
        nx = jnp.clip(ax + dx, 1, W - 2)
        ny = jnp.clip(ay + dy, 1, H - 2)

        cp = pltpu.async_copy(grids_hbm.at[:, :, pl.ds(e0, LANES)], slab_v,
                              sem)
        cp.wait()

        tcf = plsc.load_gather(slab_v, [ny, nx, lane])
        blocked = (tcf * 4.0).astype(jnp.int32) == 1
        fx = jnp.where(blocked, ax, nx)
        fy = jnp.where(blocked, ay, ny)
        cur = plsc.load_gather(slab_v, [fy, fx, lane])
        lc = (cur * 4.0).astype(jnp.int32)
        food = lc == 2
        poison = lc == 3
        reward = jnp.where(food, 10.0, 0.0) - jnp.where(poison, 20.0, 0.0) - 0.1
        enc = (en + reward) / 100.0
        centerval = jnp.where(food | poison, 0.0, cur)

        col0 = fx - 5

        @pl.loop(0, VIEW)
        def _(k):
            row = fy + (k - 5)
            rin = (row >= 0) & (row <= H - 1)
            rowc = jnp.clip(row, 0, H - 1)
            for j in range(VIEW):
                p = k * VIEW + j
                col = col0 + j
                inb = rin & (col >= 0) & (col <= W - 1)
                colc = jnp.clip(col, 0, W - 1)
                v = plsc.load_gather(slab_v, [rowc, colc, lane])
                patch = jnp.where(inb, v, 0.3)
                cell = (patch * 4.0).astype(jnp.int32)
                out_v[p, :] = patch
                out_v[PATCH + p, :] = jnp.where(cell == 1, 1.0, 0.0)
                out_v[2 * PATCH + p, :] = jnp.where(cell == 2, 1.0, 0.0)
                out_v[3 * PATCH + p, :] = jnp.where(cell == 3, 1.0, 0.0)
                out_v[4 * PATCH + p, :] = enc

        ccell = (centerval * 4.0).astype(jnp.int32)
        oc = 5 * VIEW + 5
        out_v[oc, :] = centerval
        out_v[PATCH + oc, :] = jnp.where(ccell == 1, 1.0, 0.0)
        out_v[2 * PATCH + oc, :] = jnp.where(ccell == 2, 1.0, 0.0)
        out_v[3 * PATCH + oc, :] = jnp.where(ccell == 3, 1.0, 0.0)

        pltpu.sync_copy(out_v, out_hbm.at[:, pl.ds(e0, LANES)])


def kernel(grids, agent_energy, actions, agent_x, agent_y):
    dx16 = jnp.array([0, 0, 0, -1, 1, -1, -1, 1, 1, 0, 0, 0, 0, 0, 0, 0],
                     jnp.int32)
    dy16 = jnp.array([0, -1, 1, 0, 0, -1, 1, -1, 1, 0, 0, 0, 0, 0, 0, 0],
                     jnp.int32)

    sc_fn = pl.kernel(
        _sc_body,
        out_type=jax.ShapeDtypeStruct((OUT_PER_ENV, N_ENVS), jnp.float32),
        mesh=plsc.VectorSubcoreMesh(core_axis_name="c", subcore_axis_name="s"),
        compiler_params=pltpu.CompilerParams(needs_layout_passes=False,
                                             use_tc_tiling_on_sc=False),
        scratch_types=[
            pltpu.VMEM((H, W, LANES), jnp.float32),
            pltpu.VMEM((OUT_PER_ENV, LANES), jnp.float32),
            pltpu.VMEM((EPW,), jnp.float32),
            pltpu.VMEM((EPW,), jnp.int32),
            pltpu.VMEM((EPW,), jnp.int32),
            pltpu.VMEM((EPW,), jnp.int32),
            pltpu.VMEM((LANES,), jnp.int32),
            pltpu.VMEM((LANES,), jnp.int32),
            pltpu.SemaphoreType.DMA,
        ],
    )
    flat = sc_fn(grids.transpose(1, 2, 0), agent_energy,
                 actions.astype(jnp.int32),
                 agent_x.astype(jnp.int32),
                 agent_y.astype(jnp.int32),
                 dx16, dy16)
    return flat.reshape(5, VIEW, VIEW, N_ENVS).transpose(3, 0, 1, 2)


# speedup vs baseline: 2.4280x; 1.5285x over previous
import jax
import jax.numpy as jnp
from jax import lax
from jax.experimental import pallas as pl
from jax.experimental.pallas import tpu as pltpu
from jax.experimental.pallas import tpu_sc as plsc

N_ENVS = 4096
H = 64
W = 64
VIEW = 11
NC = 2
NS = 16
LANES = 16
NW = NC * NS
EPW = N_ENVS // NW
G = EPW // LANES
OUT_PER_ENV = 5 * VIEW * VIEW
PATCH = VIEW * VIEW


NROWS = 13
Y0MAX = H - NROWS


def _sc_body(grids_hbm, en_hbm, act_hbm, ax_hbm, ay_hbm, dx_hbm, dy_hbm,
             out_hbm, slab_v, out_v, en_v, act_v, ax_v, ay_v, dx_v,
             dy_v, sem):
    wid = lax.axis_index("s") * NC + lax.axis_index("c")
    base = pl.multiple_of(wid * EPW, EPW)

    pltpu.sync_copy(en_hbm.at[pl.ds(base, EPW)], en_v)
    pltpu.sync_copy(act_hbm.at[pl.ds(base, EPW)], act_v)
    pltpu.sync_copy(ax_hbm.at[pl.ds(base, EPW)], ax_v)
    pltpu.sync_copy(ay_hbm.at[pl.ds(base, EPW)], ay_v)
    pltpu.sync_copy(dx_hbm, dx_v)
    pltpu.sync_copy(dy_hbm, dy_v)

    lane = lax.iota(jnp.int32, LANES)

    @pl.loop(0, G)
    def _(g):
        off = g * LANES
        e0 = pl.multiple_of(base + off, LANES)

        agx = ax_v[pl.ds(off, LANES)]
        agy = ay_v[pl.ds(off, LANES)]
        acts = act_v[pl.ds(off, LANES)]
        en = en_v[pl.ds(off, LANES)]

        ax = jnp.clip(agx, 1, W - 2)
        ay = jnp.clip(agy, 1, H - 2)
        dx = plsc.load_gather(dx_v, [acts])
        dy = plsc.load_gather(dy_v, [acts])
        nx = jnp.clip(ax + dx, 1, W - 2)
        ny = jnp.clip(ay + dy, 1, H - 2)

        cp = pltpu.async_copy(
            grids_hbm.at[:, :, wid, :, pl.ds(g * LANES, LANES)], slab_v, sem)
        cp.wait()

        tcf = plsc.load_gather(slab_v, [ny, nx >> 3, nx & 7, lane])
        blocked = (tcf * 4.0).astype(jnp.int32) == 1
        fx = jnp.where(blocked, ax, nx)
        fy = jnp.where(blocked, ay, ny)
        cur = plsc.load_gather(slab_v, [fy, fx >> 3, fx & 7, lane])
        lc = (cur * 4.0).astype(jnp.int32)
        food = lc == 2
        poison = lc == 3
        reward = jnp.where(food, 10.0, 0.0) - jnp.where(poison, 20.0, 0.0) - 0.1
        enc = (en + reward) / 100.0
        centerval = jnp.where(food | poison, 0.0, cur)

        col0 = fx - 5

        @pl.loop(0, VIEW)
        def _(k):
            row = fy + (k - 5)
            rin = (row >= 0) & (row <= H - 1)
            rowc = jnp.clip(row, 0, H - 1)
            for j in range(VIEW):
                p = k * VIEW + j
                col = col0 + j
                inb = rin & (col >= 0) & (col <= W - 1)
                colc = jnp.clip(col, 0, W - 1)
                v = plsc.load_gather(slab_v, [rowc, colc >> 3, colc & 7,
                                              lane])
                patch = jnp.where(inb, v, 0.3)
                cell = (patch * 4.0).astype(jnp.int32)
                out_v[p, :] = patch
                out_v[PATCH + p, :] = jnp.where(cell == 1, 1.0, 0.0)
                out_v[2 * PATCH + p, :] = jnp.where(cell == 2, 1.0, 0.0)
                out_v[3 * PATCH + p, :] = jnp.where(cell == 3, 1.0, 0.0)
                out_v[4 * PATCH + p, :] = enc

        ccell = (centerval * 4.0).astype(jnp.int32)
        oc = 5 * VIEW + 5
        out_v[oc, :] = centerval
        out_v[PATCH + oc, :] = jnp.where(ccell == 1, 1.0, 0.0)
        out_v[2 * PATCH + oc, :] = jnp.where(ccell == 2, 1.0, 0.0)
        out_v[3 * PATCH + oc, :] = jnp.where(ccell == 3, 1.0, 0.0)

        pltpu.sync_copy(out_v, out_hbm.at[:, pl.ds(e0, LANES)])


def kernel(grids, agent_energy, actions, agent_x, agent_y):
    dx16 = jnp.array([0, 0, 0, -1, 1, -1, -1, 1, 1, 0, 0, 0, 0, 0, 0, 0],
                     jnp.int32)
    dy16 = jnp.array([0, -1, 1, 0, 0, -1, 1, -1, 1, 0, 0, 0, 0, 0, 0, 0],
                     jnp.int32)

    sc_fn = pl.kernel(
        _sc_body,
        out_type=jax.ShapeDtypeStruct((OUT_PER_ENV, N_ENVS), jnp.float32),
        mesh=plsc.VectorSubcoreMesh(core_axis_name="c", subcore_axis_name="s"),
        compiler_params=pltpu.CompilerParams(needs_layout_passes=False,
                                             use_tc_tiling_on_sc=False),
        scratch_types=[
            pltpu.VMEM((H, W // 8, 8, LANES), jnp.float32),
            pltpu.VMEM((OUT_PER_ENV, LANES), jnp.float32),
            pltpu.VMEM((EPW,), jnp.float32),
            pltpu.VMEM((EPW,), jnp.int32),
            pltpu.VMEM((EPW,), jnp.int32),
            pltpu.VMEM((EPW,), jnp.int32),
            pltpu.VMEM((LANES,), jnp.int32),
            pltpu.VMEM((LANES,), jnp.int32),
            pltpu.SemaphoreType.DMA,
        ],
    )
    grids_b = (grids.transpose(1, 2, 0)
               .reshape(H, W // 8, 8, N_ENVS // 128, 128)
               .transpose(0, 1, 3, 2, 4))
    flat = sc_fn(grids_b, agent_energy,
                 actions.astype(jnp.int32),
                 agent_x.astype(jnp.int32),
                 agent_y.astype(jnp.int32),
                 dx16, dy16)
    return flat.reshape(5, VIEW, VIEW, N_ENVS).transpose(3, 0, 1, 2)


# speedup vs baseline: 2.4761x; 1.0198x over previous
import jax
import jax.numpy as jnp
from jax import lax
from jax.experimental import pallas as pl
from jax.experimental.pallas import tpu as pltpu
from jax.experimental.pallas import tpu_sc as plsc

N_ENVS = 4096
H = 64
W = 64
VIEW = 11
NC = 2
NS = 16
LANES = 16
NW = NC * NS
EPW = N_ENVS // NW
G = EPW // LANES
OUT_PER_ENV = 5 * VIEW * VIEW
PATCH = VIEW * VIEW


NROWS = 13
Y0MAX = H - NROWS


def _sc_body(grids_hbm, en_hbm, act_hbm, ax_hbm, ay_hbm, dx_hbm, dy_hbm,
             out_hbm, slab_v, out_v, en_v, act_v, ax_v, ay_v, dx_v,
             dy_v, sem, osem):
    wid = lax.axis_index("s") * NC + lax.axis_index("c")
    base = pl.multiple_of(wid * EPW, EPW)

    pltpu.sync_copy(en_hbm.at[pl.ds(base, EPW)], en_v)
    pltpu.sync_copy(act_hbm.at[pl.ds(base, EPW)], act_v)
    pltpu.sync_copy(ax_hbm.at[pl.ds(base, EPW)], ax_v)
    pltpu.sync_copy(ay_hbm.at[pl.ds(base, EPW)], ay_v)
    pltpu.sync_copy(dx_hbm, dx_v)
    pltpu.sync_copy(dy_hbm, dy_v)

    lane = lax.iota(jnp.int32, LANES)

    @pl.loop(0, G)
    def _(g):
        off = g * LANES
        e0 = pl.multiple_of(base + off, LANES)

        agx = ax_v[pl.ds(off, LANES)]
        agy = ay_v[pl.ds(off, LANES)]
        acts = act_v[pl.ds(off, LANES)]
        en = en_v[pl.ds(off, LANES)]

        ax = jnp.clip(agx, 1, W - 2)
        ay = jnp.clip(agy, 1, H - 2)
        dx = plsc.load_gather(dx_v, [acts])
        dy = plsc.load_gather(dy_v, [acts])
        nx = jnp.clip(ax + dx, 1, W - 2)
        ny = jnp.clip(ay + dy, 1, H - 2)

        cp = pltpu.async_copy(
            grids_hbm.at[:, :, wid, :, pl.ds(g * LANES, LANES)], slab_v, sem)
        cp.wait()

        tcf = plsc.load_gather(slab_v, [ny, nx >> 3, nx & 7, lane])
        blocked = (tcf * 4.0).astype(jnp.int32) == 1
        fx = jnp.where(blocked, ax, nx)
        fy = jnp.where(blocked, ay, ny)
        @pl.when(g > 0)
        def _():
            pltpu.make_async_copy(out_v, out_hbm.at[:, pl.ds(e0, LANES)],
                                  osem).wait()

        cur = plsc.load_gather(slab_v, [fy, fx >> 3, fx & 7, lane])
        lc = (cur * 4.0).astype(jnp.int32)
        food = lc == 2
        poison = lc == 3
        reward = jnp.where(food, 10.0, 0.0) - jnp.where(poison, 20.0, 0.0) - 0.1
        enc = (en + reward) / 100.0
        centerval = jnp.where(food | poison, 0.0, cur)

        col0 = fx - 5

        @pl.loop(0, VIEW)
        def _(k):
            row = fy + (k - 5)
            rin = (row >= 0) & (row <= H - 1)
            rowc = jnp.clip(row, 0, H - 1)
            for j in range(VIEW):
                p = k * VIEW + j
                col = col0 + j
                inb = rin & (col >= 0) & (col <= W - 1)
                colc = jnp.clip(col, 0, W - 1)
                v = plsc.load_gather(slab_v, [rowc, colc >> 3, colc & 7,
                                              lane])
                patch = jnp.where(inb, v, 0.3)
                cell = (patch * 4.0).astype(jnp.int32)
                out_v[p, :] = patch
                out_v[PATCH + p, :] = jnp.where(cell == 1, 1.0, 0.0)
                out_v[2 * PATCH + p, :] = jnp.where(cell == 2, 1.0, 0.0)
                out_v[3 * PATCH + p, :] = jnp.where(cell == 3, 1.0, 0.0)
                out_v[4 * PATCH + p, :] = enc

        ccell = (centerval * 4.0).astype(jnp.int32)
        oc = 5 * VIEW + 5
        out_v[oc, :] = centerval
        out_v[PATCH + oc, :] = jnp.where(ccell == 1, 1.0, 0.0)
        out_v[2 * PATCH + oc, :] = jnp.where(ccell == 2, 1.0, 0.0)
        out_v[3 * PATCH + oc, :] = jnp.where(ccell == 3, 1.0, 0.0)

        pltpu.async_copy(out_v, out_hbm.at[:, pl.ds(e0, LANES)], osem)

    last = pl.multiple_of(base + (G - 1) * LANES, LANES)
    pltpu.make_async_copy(out_v, out_hbm.at[:, pl.ds(last, LANES)],
                          osem).wait()


def kernel(grids, agent_energy, actions, agent_x, agent_y):
    dx16 = jnp.array([0, 0, 0, -1, 1, -1, -1, 1, 1, 0, 0, 0, 0, 0, 0, 0],
                     jnp.int32)
    dy16 = jnp.array([0, -1, 1, 0, 0, -1, 1, -1, 1, 0, 0, 0, 0, 0, 0, 0],
                     jnp.int32)

    sc_fn = pl.kernel(
        _sc_body,
        out_type=jax.ShapeDtypeStruct((OUT_PER_ENV, N_ENVS), jnp.float32),
        mesh=plsc.VectorSubcoreMesh(core_axis_name="c", subcore_axis_name="s"),
        compiler_params=pltpu.CompilerParams(needs_layout_passes=False,
                                             use_tc_tiling_on_sc=False),
        scratch_types=[
            pltpu.VMEM((H, W // 8, 8, LANES), jnp.float32),
            pltpu.VMEM((OUT_PER_ENV, LANES), jnp.float32),
            pltpu.VMEM((EPW,), jnp.float32),
            pltpu.VMEM((EPW,), jnp.int32),
            pltpu.VMEM((EPW,), jnp.int32),
            pltpu.VMEM((EPW,), jnp.int32),
            pltpu.VMEM((LANES,), jnp.int32),
            pltpu.VMEM((LANES,), jnp.int32),
            pltpu.SemaphoreType.DMA,
            pltpu.SemaphoreType.DMA,
        ],
    )
    grids_b = (grids.transpose(1, 2, 0)
               .reshape(H, W // 8, 8, N_ENVS // 128, 128)
               .transpose(0, 1, 3, 2, 4))
    flat = sc_fn(grids_b, agent_energy,
                 actions.astype(jnp.int32),
                 agent_x.astype(jnp.int32),
                 agent_y.astype(jnp.int32),
                 dx16, dy16)
    return flat.reshape(5, VIEW, VIEW, N_ENVS).transpose(3, 0, 1, 2)


# speedup vs baseline: 2.4842x; 1.0033x over previous
import jax
import jax.numpy as jnp
from jax import lax
from jax.experimental import pallas as pl
from jax.experimental.pallas import tpu as pltpu
from jax.experimental.pallas import tpu_sc as plsc

N_ENVS = 4096
H = 64
W = 64
VIEW = 11
NC = 2
NS = 16
LANES = 16
NW = NC * NS
EPW = N_ENVS // NW
G = EPW // LANES
OUT_PER_ENV = 5 * VIEW * VIEW
PATCH = VIEW * VIEW


NROWS = 13
Y0MAX = H - NROWS


def _sc_body(grids_hbm, en_hbm, act_hbm, ax_hbm, ay_hbm, dx_hbm, dy_hbm,
             out_hbm, slab_v, out_v, en_v, act_v, ax_v, ay_v, dx_v,
             dy_v, sem, osem):
    wid = lax.axis_index("s") * NC + lax.axis_index("c")
    base = pl.multiple_of(wid * EPW, EPW)

    pltpu.sync_copy(en_hbm.at[pl.ds(base, EPW)], en_v)
    pltpu.sync_copy(act_hbm.at[pl.ds(base, EPW)], act_v)
    pltpu.sync_copy(ax_hbm.at[pl.ds(base, EPW)], ax_v)
    pltpu.sync_copy(ay_hbm.at[pl.ds(base, EPW)], ay_v)
    pltpu.sync_copy(dx_hbm, dx_v)
    pltpu.sync_copy(dy_hbm, dy_v)

    lane = lax.iota(jnp.int32, LANES)

    @pl.loop(0, G)
    def _(g):
        off = g * LANES
        e0 = pl.multiple_of(base + off, LANES)

        agx = ax_v[pl.ds(off, LANES)]
        agy = ay_v[pl.ds(off, LANES)]
        acts = act_v[pl.ds(off, LANES)]
        en = en_v[pl.ds(off, LANES)]

        ax = jnp.clip(agx, 1, W - 2)
        ay = jnp.clip(agy, 1, H - 2)
        dx = plsc.load_gather(dx_v, [acts])
        dy = plsc.load_gather(dy_v, [acts])
        nx = jnp.clip(ax + dx, 1, W - 2)
        ny = jnp.clip(ay + dy, 1, H - 2)

        cp = pltpu.async_copy(
            grids_hbm.at[:, :, wid, :, pl.ds(g * LANES, LANES)], slab_v, sem)
        cp.wait()

        tcf = plsc.load_gather(slab_v, [ny, nx >> 3, nx & 7, lane])
        blocked = (tcf * 4.0).astype(jnp.int32) == 1
        fx = jnp.where(blocked, ax, nx)
        fy = jnp.where(blocked, ay, ny)
        @pl.when(g > 0)
        def _():
            pltpu.make_async_copy(out_v, out_hbm.at[:, pl.ds(e0, LANES)],
                                  osem).wait()

        cur = plsc.load_gather(slab_v, [fy, fx >> 3, fx & 7, lane])
        lc = (cur * 4.0).astype(jnp.int32)
        food = lc == 2
        poison = lc == 3
        reward = jnp.where(food, 10.0, 0.0) - jnp.where(poison, 20.0, 0.0) - 0.1
        enc = (en + reward) / 100.0
        centerval = jnp.where(food | poison, 0.0, cur)

        col0 = fx - 5

        cols = []
        for j in range(VIEW):
            col = col0 + j
            cin = (col >= 0) & (col <= W - 1)
            colc = jnp.clip(col, 0, W - 1)
            cols.append((cin, colc >> 3, colc & 7))

        @pl.loop(0, VIEW)
        def _(k):
            row = fy + (k - 5)
            rin = (row >= 0) & (row <= H - 1)
            rowc = jnp.clip(row, 0, H - 1)
            for j in range(VIEW):
                p = k * VIEW + j
                cin, cxr, cxs = cols[j]
                inb = rin & cin
                v = plsc.load_gather(slab_v, [rowc, cxr, cxs, lane])
                patch = jnp.where(inb, v, 0.3)
                cell = (patch * 4.0).astype(jnp.int32)
                out_v[p, :] = patch
                out_v[PATCH + p, :] = jnp.where(cell == 1, 1.0, 0.0)
                out_v[2 * PATCH + p, :] = jnp.where(cell == 2, 1.0, 0.0)
                out_v[3 * PATCH + p, :] = jnp.where(cell == 3, 1.0, 0.0)
                out_v[4 * PATCH + p, :] = enc

        ccell = (centerval * 4.0).astype(jnp.int32)
        oc = 5 * VIEW + 5
        out_v[oc, :] = centerval
        out_v[PATCH + oc, :] = jnp.where(ccell == 1, 1.0, 0.0)
        out_v[2 * PATCH + oc, :] = jnp.where(ccell == 2, 1.0, 0.0)
        out_v[3 * PATCH + oc, :] = jnp.where(ccell == 3, 1.0, 0.0)

        pltpu.async_copy(out_v, out_hbm.at[:, pl.ds(e0, LANES)], osem)

    last = pl.multiple_of(base + (G - 1) * LANES, LANES)
    pltpu.make_async_copy(out_v, out_hbm.at[:, pl.ds(last, LANES)],
                          osem).wait()


def kernel(grids, agent_energy, actions, agent_x, agent_y):
    dx16 = jnp.array([0, 0, 0, -1, 1, -1, -1, 1, 1, 0, 0, 0, 0, 0, 0, 0],
                     jnp.int32)
    dy16 = jnp.array([0, -1, 1, 0, 0, -1, 1, -1, 1, 0, 0, 0, 0, 0, 0, 0],
                     jnp.int32)

    sc_fn = pl.kernel(
        _sc_body,
        out_type=jax.ShapeDtypeStruct((OUT_PER_ENV, N_ENVS), jnp.float32),
        mesh=plsc.VectorSubcoreMesh(core_axis_name="c", subcore_axis_name="s"),
        compiler_params=pltpu.CompilerParams(needs_layout_passes=False,
                                             use_tc_tiling_on_sc=False),
        scratch_types=[
            pltpu.VMEM((H, W // 8, 8, LANES), jnp.float32),
            pltpu.VMEM((OUT_PER_ENV, LANES), jnp.float32),
            pltpu.VMEM((EPW,), jnp.float32),
            pltpu.VMEM((EPW,), jnp.int32),
            pltpu.VMEM((EPW,), jnp.int32),
            pltpu.VMEM((EPW,), jnp.int32),
            pltpu.VMEM((LANES,), jnp.int32),
            pltpu.VMEM((LANES,), jnp.int32),
            pltpu.SemaphoreType.DMA,
            pltpu.SemaphoreType.DMA,
        ],
    )
    grids_b = (grids.transpose(1, 2, 0)
               .reshape(H, W // 8, 8, N_ENVS // 128, 128)
               .transpose(0, 1, 3, 2, 4))
    flat = sc_fn(grids_b, agent_energy,
                 actions.astype(jnp.int32),
                 agent_x.astype(jnp.int32),
                 agent_y.astype(jnp.int32),
                 dx16, dy16)
    return flat.reshape(5, VIEW, VIEW, N_ENVS).transpose(3, 0, 1, 2)
